# baseline (speedup 1.0000x reference)
import functools

import jax
import jax.numpy as jnp
from jax import lax
from jax.experimental import pallas as pl
from jax.experimental.pallas import tpu as pltpu
from jax.experimental.pallas import tpu_sc as plsc

_NC = 2
_NS = 16
_CHUNK = 80
_UNROLL = 3


def _prep_body(x_ref, wcat_ref, wad_ref, ea_ref, xa_ref, adst_ref, easum_ref):
    i = pl.program_id(0)
    xv = x_ref[...]
    xa_ref[...] = jnp.dot(xv, wcat_ref[...], preferred_element_type=jnp.float32)
    adst_ref[...] = jnp.dot(xv, wad_ref[...], preferred_element_type=jnp.float32)

    @pl.when(i == 0)
    def _():
        easum_ref[...] = jnp.zeros_like(easum_ref)

    easum_ref[...] += jnp.sum(ea_ref[...])[None, None]


def _dyn_gather(v, idx):
    return lax.gather(
        v, idx[:, None],
        lax.GatherDimensionNumbers(
            offset_dims=(), collapsed_slice_dims=(0,), start_index_map=(0,)),
        slice_sizes=(1,),
        mode=lax.GatherScatterMode.PROMISE_IN_BOUNDS)


def _chunks_per_tile(E):
    nw = _NC * _NS
    cpt = -(-E // (_CHUNK * nw))
    return -(-cpt // _UNROLL) * _UNROLL


def _make_sc_edge(N, E, Dp):
    cpt = _chunks_per_tile(E)
    cpt0 = int(round(cpt * 2 * 0.41 / _UNROLL)) * _UNROLL
    cpt1 = 2 * cpt - cpt0
    rows_per_tile = -(-N // (_NS * 8)) * 8
    Npad = rows_per_tile * _NS

    mesh = plsc.VectorSubcoreMesh(core_axis_name="c", subcore_axis_name="s")

    idx_t = pltpu.VMEM((_CHUNK,), jnp.int32)
    ea_t = pltpu.VMEM((_CHUNK,), jnp.float32)
    rows_t = pltpu.VMEM((_CHUNK, Dp), jnp.float32)
    adr_t = pltpu.VMEM((_CHUNK, 16), jnp.float32)

    @functools.partial(
        pl.kernel,
        out_type=jax.ShapeDtypeStruct((_NC, Npad, Dp), jnp.float32),
        mesh=mesh,
        compiler_params=pltpu.CompilerParams(
            use_tc_tiling_on_sc=False, needs_layout_passes=False),
        scratch_types=[
            [idx_t] * 3, [idx_t] * 3, [ea_t] * 3,
            [rows_t] * 3,
            [adr_t] * 3,
            pltpu.VMEM((16,), jnp.float32),
            pltpu.VMEM_SHARED((Npad, Dp), jnp.float32),
            [pltpu.SemaphoreType.DMA] * 3,
            [pltpu.SemaphoreType.DMA] * 3,
            pltpu.SemaphoreType.DMA,
        ],
    )
    def sc_fn(xa, adst16, srcA, dstA, eaA, c16, zrows, out,
              srcv, dstv, eav, rows, adr, cbuf, acc, sem_i, sem_g, sem_s):
        cid = lax.axis_index("c")
        sid = lax.axis_index("s")
        nmine = jnp.where(cid == 0, cpt0, cpt1)
        last = nmine - 1
        start = jnp.where(cid == 0, sid * cpt0, _NS * cpt0 + sid * cpt1)

        pltpu.sync_copy(zrows, acc.at[pl.ds(sid * rows_per_tile, rows_per_tile)])
        pltpu.sync_copy(c16, cbuf)
        plsc.subcore_barrier()

        c2v = cbuf[...]
        iot = lax.iota(jnp.int32, 16)
        row_off = iot >> 3
        colc = iot & 7

        def issue_idx(q, j, s):
            base = (start + q) * _CHUNK
            pltpu.async_copy(srcA.at[pl.ds(base, _CHUNK)], srcv[j], sem_i[s])
            pltpu.async_copy(dstA.at[pl.ds(base, _CHUNK)], dstv[j], sem_i[s])
            pltpu.async_copy(eaA.at[pl.ds(base, _CHUNK)], eav[j], sem_i[s])

        def wait_idx(j, s):
            pltpu.make_async_copy(srcA.at[pl.ds(0, _CHUNK)], srcv[j], sem_i[s]).wait()
            pltpu.make_async_copy(dstA.at[pl.ds(0, _CHUNK)], dstv[j], sem_i[s]).wait()
            pltpu.make_async_copy(eaA.at[pl.ds(0, _CHUNK)], eav[j], sem_i[s]).wait()

        def issue_gather(j, r, s):
            pltpu.async_copy(xa.at[srcv[j]], rows[r], sem_g[s])
            pltpu.async_copy(adst16.at[dstv[j]], adr[r], sem_g[s])

        def wait_gather(j, r, s):
            pltpu.make_async_copy(xa.at[srcv[j]], rows[r], sem_g[s]).wait()
            pltpu.make_async_copy(adst16.at[dstv[j]], adr[r], sem_g[s]).wait()

        def compute(r, j):
            rows_b, adr_b, eav_b = rows[r], adr[r], eav[j]

            @plsc.parallel_loop(0, _CHUNK // 2, unroll=2)
            def pair_body(p):
                e0 = 2 * p
                r2 = jnp.full((16,), e0, jnp.int32) + row_off
                a1 = plsc.load_gather(rows_b, [r2, colc + 128])
                a2 = plsc.load_gather(adr_b, [r2, colc])
                eb = plsc.load_gather(eav_b, [r2])
                al = a1 + a2 + eb * c2v
                ex = jnp.exp(jnp.maximum(al, 0.2 * al))
                plsc.store_scatter(rows_b, [r2, colc + 128], ex)
                for h in range(8):
                    s0 = _dyn_gather(ex, jnp.full((16,), h, jnp.int32))
                    s1 = _dyn_gather(ex, jnp.full((16,), 8 + h, jnp.int32))
                    rows_b[e0, pl.ds(16 * h, 16)] = (
                        rows_b[e0, pl.ds(16 * h, 16)] * s0)
                    rows_b[e0 + 1, pl.ds(16 * h, 16)] = (
                        rows_b[e0 + 1, pl.ds(16 * h, 16)] * s1)

        def issue_scatter(r, j):
            pltpu.async_copy(rows[r], acc.at[dstv[j]], sem_s, add=True)

        def wait_scatter(r, j):
            pltpu.make_async_copy(rows[r], acc.at[dstv[j]], sem_s).wait()

        issue_idx(0, 0, 0)
        wait_idx(0, 0)
        issue_gather(0, 0, 0)
        issue_idx(1, 1, 1)

        def outer(g, carry):
            for b in range(_UNROLL):
                q = _UNROLL * g + b
                r0, r1, r2_ = b % 3, (b + 1) % 3, (b + 2) % 3

                wait_gather(r0, r0, r0)

                @pl.when(q < last)
                def _():
                    wait_idx(r1, r1)
                    issue_gather(r1, r1, r1)

                compute(r0, r0)

                @pl.when(q > 0)
                def _():
                    wait_scatter(r2_, r2_)

                issue_scatter(r0, r0)

                @pl.when(q + 2 <= last)
                def _():
                    issue_idx(q + 2, r2_, r2_)
            return carry

        lax.fori_loop(0, nmine // _UNROLL, outer, 0)
        wait_scatter((_UNROLL - 1) % 3, (_UNROLL - 1) % 3)
        plsc.subcore_barrier()
        pltpu.sync_copy(
            acc.at[pl.ds(sid * rows_per_tile, rows_per_tile)],
            out.at[cid, pl.ds(sid * rows_per_tile, rows_per_tile)])

    return sc_fn


def _epi_body(num_ref, x_ref, xa_ref, easum_ref,
              c8_ref, psel_ref, padd_ref, eexp_ref, bias_ref, g_ref, b_ref,
              o_ref, *, inv_e):
    n144 = num_ref[0] + num_ref[1]
    xa = xa_ref[...]
    xp = xa[:, :128]
    me = easum_ref[...] * inv_e
    asum = jnp.dot(xa, padd_ref[...], preferred_element_type=jnp.float32)
    al = asum + me * c8_ref[...]
    ex8 = jnp.exp(jnp.maximum(al, 0.2 * al))
    den8 = jnp.dot(n144, psel_ref[...], preferred_element_type=jnp.float32) + ex8
    eexp = eexp_ref[...]
    num = (n144[:, :128]
           + xp * jnp.dot(ex8, eexp, preferred_element_type=jnp.float32))
    den = jnp.dot(den8, eexp, preferred_element_type=jnp.float32)
    out = num / (den + 1e-16) + bias_ref[...]
    h = out + x_ref[...]
    mu = jnp.mean(h, axis=1, keepdims=True)
    hc = h - mu
    var = jnp.mean(hc * hc, axis=1, keepdims=True)
    o_ref[...] = jax.nn.relu(hc / jnp.sqrt(var + 1e-5) * g_ref[...] + b_ref[...])


def kernel(x, edge_index, edge_attr, W, W_edge, att_src, att_dst, att_edge,
           bias, ln_g, ln_b):
    N, D = x.shape
    E = edge_attr.shape[0]
    H, Ch = att_src.shape
    Dp = D + 2 * H

    As = (W.reshape(D, H, Ch) * att_src[None]).sum(-1)
    Ad = (W.reshape(D, H, Ch) * att_dst[None]).sum(-1)
    c8 = (W_edge.reshape(H, Ch) * att_edge).sum(-1)
    Wcat = jnp.concatenate([W, As, Ad], axis=1)
    Wad = jnp.concatenate([Ad, jnp.zeros((D, H), jnp.float32)], axis=1)
    c16 = jnp.concatenate([c8, c8])
    eye = jnp.eye(H, dtype=jnp.float32)
    zpad = jnp.zeros((D, H), jnp.float32)
    psel = jnp.concatenate([zpad, eye, jnp.zeros((H, H), jnp.float32)], axis=0)
    padd = jnp.concatenate([zpad, eye, eye], axis=0)
    eexp = jnp.repeat(eye, Ch, axis=1)

    BN = 1000
    grid = N // BN
    EB = E // grid

    xa, adst16, easum = pl.pallas_call(
        _prep_body,
        grid=(grid,),
        in_specs=[
            pl.BlockSpec((BN, D), lambda i: (i, 0)),
            pl.BlockSpec((D, Dp), lambda i: (0, 0)),
            pl.BlockSpec((D, 16), lambda i: (0, 0)),
            pl.BlockSpec((1, 1, EB), lambda i: (i, 0, 0)),
        ],
        out_specs=[
            pl.BlockSpec((BN, Dp), lambda i: (i, 0)),
            pl.BlockSpec((BN, 16), lambda i: (i, 0)),
            pl.BlockSpec((1, 1), lambda i: (0, 0)),
        ],
        out_shape=[
            jax.ShapeDtypeStruct((N, Dp), jnp.float32),
            jax.ShapeDtypeStruct((N, 16), jnp.float32),
            jax.ShapeDtypeStruct((1, 1), jnp.float32),
        ],
    )(x, Wcat, Wad, edge_attr.reshape(grid, 1, EB))

    rows_per_tile = -(-N // (_NS * 8)) * 8
    Npad = rows_per_tile * _NS
    zrows = jnp.zeros((rows_per_tile, Dp), jnp.float32)
    Epad = _NC * _NS * _chunks_per_tile(E) * _CHUNK
    npadE = Epad - E
    src_p = jnp.concatenate([edge_index[0], jnp.zeros((npadE,), jnp.int32)])
    dst_p = jnp.concatenate(
        [edge_index[1], jnp.full((npadE,), Npad - 1, jnp.int32)])
    ea_p = jnp.concatenate([edge_attr, jnp.zeros((npadE,), jnp.float32)])
    num2 = _make_sc_edge(N, E, Dp)(xa, adst16, src_p, dst_p, ea_p, c16, zrows)

    out = pl.pallas_call(
        functools.partial(_epi_body, inv_e=1.0 / E),
        grid=(grid,),
        in_specs=[
            pl.BlockSpec((_NC, BN, Dp), lambda i: (0, i, 0)),
            pl.BlockSpec((BN, D), lambda i: (i, 0)),
            pl.BlockSpec((BN, Dp), lambda i: (i, 0)),
            pl.BlockSpec((1, 1), lambda i: (0, 0)),
            pl.BlockSpec((1, H), lambda i: (0, 0)),
            pl.BlockSpec((Dp, H), lambda i: (0, 0)),
            pl.BlockSpec((Dp, H), lambda i: (0, 0)),
            pl.BlockSpec((H, D), lambda i: (0, 0)),
            pl.BlockSpec((1, D), lambda i: (0, 0)),
            pl.BlockSpec((1, D), lambda i: (0, 0)),
            pl.BlockSpec((1, D), lambda i: (0, 0)),
        ],
        out_specs=pl.BlockSpec((BN, D), lambda i: (i, 0)),
        out_shape=jax.ShapeDtypeStruct((N, D), jnp.float32),
    )(num2, x, xa, easum, c8.reshape(1, H), psel, padd, eexp,
      bias.reshape(1, D), ln_g.reshape(1, D), ln_b.reshape(1, D))
    return out


# speedup vs baseline: 1.1110x; 1.1110x over previous
import functools

import jax
import jax.numpy as jnp
from jax import lax
from jax.experimental import pallas as pl
from jax.experimental.pallas import tpu as pltpu
from jax.experimental.pallas import tpu_sc as plsc

_NC = 2
_NS = 16
_CHUNK = 80
_UNROLL = 3


def _prep_body(x_ref, wcat_ref, wad_ref, ea_ref, xa_ref, adst_ref, easum_ref):
    i = pl.program_id(0)
    xv = x_ref[...]
    xa_ref[...] = jnp.dot(xv, wcat_ref[...], preferred_element_type=jnp.float32)
    adst_ref[...] = jnp.dot(xv, wad_ref[...], preferred_element_type=jnp.float32)

    @pl.when(i == 0)
    def _():
        easum_ref[...] = jnp.zeros_like(easum_ref)

    easum_ref[...] += jnp.sum(ea_ref[...])[None, None]


def _dyn_gather(v, idx):
    return lax.gather(
        v, idx[:, None],
        lax.GatherDimensionNumbers(
            offset_dims=(), collapsed_slice_dims=(0,), start_index_map=(0,)),
        slice_sizes=(1,),
        mode=lax.GatherScatterMode.PROMISE_IN_BOUNDS)


def _chunks_per_tile(E):
    nw = _NC * _NS
    cpt = -(-E // (_CHUNK * nw))
    return -(-cpt // _UNROLL) * _UNROLL


def _make_sc_edge(N, E, Dp):
    cpt = _chunks_per_tile(E)
    cpt0 = int(round(cpt * 2 * 0.59 / _UNROLL)) * _UNROLL
    cpt1 = 2 * cpt - cpt0
    rows_per_tile = -(-N // (_NS * 8)) * 8
    Npad = rows_per_tile * _NS

    mesh = plsc.VectorSubcoreMesh(core_axis_name="c", subcore_axis_name="s")

    idx_t = pltpu.VMEM((_CHUNK,), jnp.int32)
    ea_t = pltpu.VMEM((_CHUNK,), jnp.float32)
    rows_t = pltpu.VMEM((_CHUNK, Dp), jnp.float32)
    adr_t = pltpu.VMEM((_CHUNK, 16), jnp.float32)

    @functools.partial(
        pl.kernel,
        out_type=jax.ShapeDtypeStruct((_NC, Npad, Dp), jnp.float32),
        mesh=mesh,
        compiler_params=pltpu.CompilerParams(
            use_tc_tiling_on_sc=False, needs_layout_passes=False),
        scratch_types=[
            [idx_t] * 3, [idx_t] * 3, [ea_t] * 3,
            [rows_t] * 3,
            [adr_t] * 3,
            pltpu.VMEM((16,), jnp.float32),
            pltpu.VMEM_SHARED((Npad, Dp), jnp.float32),
            [pltpu.SemaphoreType.DMA] * 3,
            [pltpu.SemaphoreType.DMA] * 3,
            pltpu.SemaphoreType.DMA,
        ],
    )
    def sc_fn(xa, adst16, srcA, dstA, eaA, c16, zrows, out,
              srcv, dstv, eav, rows, adr, cbuf, acc, sem_i, sem_g, sem_s):
        cid = lax.axis_index("c")
        sid = lax.axis_index("s")
        nmine = jnp.where(cid == 0, cpt0, cpt1)
        last = nmine - 1
        start = jnp.where(cid == 0, sid * cpt0, _NS * cpt0 + sid * cpt1)

        pltpu.sync_copy(zrows, acc.at[pl.ds(sid * rows_per_tile, rows_per_tile)])
        pltpu.sync_copy(c16, cbuf)
        plsc.subcore_barrier()

        c2v = cbuf[...]
        iot = lax.iota(jnp.int32, 16)
        row_off = iot >> 3
        colc = iot & 7

        def issue_idx(q, j, s):
            base = (start + q) * _CHUNK
            pltpu.async_copy(srcA.at[pl.ds(base, _CHUNK)], srcv[j], sem_i[s])
            pltpu.async_copy(dstA.at[pl.ds(base, _CHUNK)], dstv[j], sem_i[s])
            pltpu.async_copy(eaA.at[pl.ds(base, _CHUNK)], eav[j], sem_i[s])

        def wait_idx(j, s):
            pltpu.make_async_copy(srcA.at[pl.ds(0, _CHUNK)], srcv[j], sem_i[s]).wait()
            pltpu.make_async_copy(dstA.at[pl.ds(0, _CHUNK)], dstv[j], sem_i[s]).wait()
            pltpu.make_async_copy(eaA.at[pl.ds(0, _CHUNK)], eav[j], sem_i[s]).wait()

        def issue_gather(j, r, s):
            pltpu.async_copy(xa.at[srcv[j]], rows[r], sem_g[s])
            pltpu.async_copy(adst16.at[dstv[j]], adr[r], sem_g[s])

        def wait_gather(j, r, s):
            pltpu.make_async_copy(xa.at[srcv[j]], rows[r], sem_g[s]).wait()
            pltpu.make_async_copy(adst16.at[dstv[j]], adr[r], sem_g[s]).wait()

        def compute(r, j):
            rows_b, adr_b, eav_b = rows[r], adr[r], eav[j]

            @plsc.parallel_loop(0, _CHUNK // 2, unroll=2)
            def pair_body(p):
                e0 = 2 * p
                r2 = jnp.full((16,), e0, jnp.int32) + row_off
                a1 = plsc.load_gather(rows_b, [r2, colc + 128])
                a2 = plsc.load_gather(adr_b, [r2, colc])
                eb = plsc.load_gather(eav_b, [r2])
                al = a1 + a2 + eb * c2v
                ex = jnp.exp(jnp.maximum(al, 0.2 * al))
                plsc.store_scatter(rows_b, [r2, colc + 128], ex)
                for h in range(8):
                    s0 = _dyn_gather(ex, jnp.full((16,), h, jnp.int32))
                    s1 = _dyn_gather(ex, jnp.full((16,), 8 + h, jnp.int32))
                    rows_b[e0, pl.ds(16 * h, 16)] = (
                        rows_b[e0, pl.ds(16 * h, 16)] * s0)
                    rows_b[e0 + 1, pl.ds(16 * h, 16)] = (
                        rows_b[e0 + 1, pl.ds(16 * h, 16)] * s1)

        def issue_scatter(r, j):
            pltpu.async_copy(rows[r], acc.at[dstv[j]], sem_s, add=True)

        def wait_scatter(r, j):
            pltpu.make_async_copy(rows[r], acc.at[dstv[j]], sem_s).wait()

        issue_idx(0, 0, 0)
        wait_idx(0, 0)
        issue_gather(0, 0, 0)
        issue_idx(1, 1, 1)

        def outer(g, carry):
            for b in range(_UNROLL):
                q = _UNROLL * g + b
                r0, r1, r2_ = b % 3, (b + 1) % 3, (b + 2) % 3

                wait_gather(r0, r0, r0)

                @pl.when(q < last)
                def _():
                    wait_idx(r1, r1)
                    issue_gather(r1, r1, r1)

                compute(r0, r0)

                @pl.when(q > 0)
                def _():
                    wait_scatter(r2_, r2_)

                issue_scatter(r0, r0)

                @pl.when(q + 2 <= last)
                def _():
                    issue_idx(q + 2, r2_, r2_)
            return carry

        lax.fori_loop(0, nmine // _UNROLL, outer, 0)
        wait_scatter((_UNROLL - 1) % 3, (_UNROLL - 1) % 3)
        plsc.subcore_barrier()
        pltpu.sync_copy(
            acc.at[pl.ds(sid * rows_per_tile, rows_per_tile)],
            out.at[cid, pl.ds(sid * rows_per_tile, rows_per_tile)])

    return sc_fn


def _epi_body(num_ref, x_ref, xa_ref, easum_ref,
              c8_ref, psel_ref, padd_ref, eexp_ref, bias_ref, g_ref, b_ref,
              o_ref, *, inv_e):
    n144 = num_ref[0] + num_ref[1]
    xa = xa_ref[...]
    xp = xa[:, :128]
    me = easum_ref[...] * inv_e
    asum = jnp.dot(xa, padd_ref[...], preferred_element_type=jnp.float32)
    al = asum + me * c8_ref[...]
    ex8 = jnp.exp(jnp.maximum(al, 0.2 * al))
    den8 = jnp.dot(n144, psel_ref[...], preferred_element_type=jnp.float32) + ex8
    eexp = eexp_ref[...]
    num = (n144[:, :128]
           + xp * jnp.dot(ex8, eexp, preferred_element_type=jnp.float32))
    den = jnp.dot(den8, eexp, preferred_element_type=jnp.float32)
    out = num / (den + 1e-16) + bias_ref[...]
    h = out + x_ref[...]
    mu = jnp.mean(h, axis=1, keepdims=True)
    hc = h - mu
    var = jnp.mean(hc * hc, axis=1, keepdims=True)
    o_ref[...] = jax.nn.relu(hc / jnp.sqrt(var + 1e-5) * g_ref[...] + b_ref[...])


def kernel(x, edge_index, edge_attr, W, W_edge, att_src, att_dst, att_edge,
           bias, ln_g, ln_b):
    N, D = x.shape
    E = edge_attr.shape[0]
    H, Ch = att_src.shape
    Dp = D + 2 * H

    As = (W.reshape(D, H, Ch) * att_src[None]).sum(-1)
    Ad = (W.reshape(D, H, Ch) * att_dst[None]).sum(-1)
    c8 = (W_edge.reshape(H, Ch) * att_edge).sum(-1)
    Wcat = jnp.concatenate([W, As, Ad], axis=1)
    Wad = jnp.concatenate([Ad, jnp.zeros((D, H), jnp.float32)], axis=1)
    c16 = jnp.concatenate([c8, c8])
    eye = jnp.eye(H, dtype=jnp.float32)
    zpad = jnp.zeros((D, H), jnp.float32)
    psel = jnp.concatenate([zpad, eye, jnp.zeros((H, H), jnp.float32)], axis=0)
    padd = jnp.concatenate([zpad, eye, eye], axis=0)
    eexp = jnp.repeat(eye, Ch, axis=1)

    BN = 1000
    grid = N // BN
    EB = E // grid

    xa, adst16, easum = pl.pallas_call(
        _prep_body,
        grid=(grid,),
        in_specs=[
            pl.BlockSpec((BN, D), lambda i: (i, 0)),
            pl.BlockSpec((D, Dp), lambda i: (0, 0)),
            pl.BlockSpec((D, 16), lambda i: (0, 0)),
            pl.BlockSpec((1, 1, EB), lambda i: (i, 0, 0)),
        ],
        out_specs=[
            pl.BlockSpec((BN, Dp), lambda i: (i, 0)),
            pl.BlockSpec((BN, 16), lambda i: (i, 0)),
            pl.BlockSpec((1, 1), lambda i: (0, 0)),
        ],
        out_shape=[
            jax.ShapeDtypeStruct((N, Dp), jnp.float32),
            jax.ShapeDtypeStruct((N, 16), jnp.float32),
            jax.ShapeDtypeStruct((1, 1), jnp.float32),
        ],
    )(x, Wcat, Wad, edge_attr.reshape(grid, 1, EB))

    rows_per_tile = -(-N // (_NS * 8)) * 8
    Npad = rows_per_tile * _NS
    zrows = jnp.zeros((rows_per_tile, Dp), jnp.float32)
    Epad = _NC * _NS * _chunks_per_tile(E) * _CHUNK
    npadE = Epad - E
    src_p = jnp.concatenate([edge_index[0], jnp.zeros((npadE,), jnp.int32)])
    dst_p = jnp.concatenate(
        [edge_index[1], jnp.full((npadE,), Npad - 1, jnp.int32)])
    ea_p = jnp.concatenate([edge_attr, jnp.zeros((npadE,), jnp.float32)])
    num2 = _make_sc_edge(N, E, Dp)(xa, adst16, src_p, dst_p, ea_p, c16, zrows)

    out = pl.pallas_call(
        functools.partial(_epi_body, inv_e=1.0 / E),
        grid=(grid,),
        in_specs=[
            pl.BlockSpec((_NC, BN, Dp), lambda i: (0, i, 0)),
            pl.BlockSpec((BN, D), lambda i: (i, 0)),
            pl.BlockSpec((BN, Dp), lambda i: (i, 0)),
            pl.BlockSpec((1, 1), lambda i: (0, 0)),
            pl.BlockSpec((1, H), lambda i: (0, 0)),
            pl.BlockSpec((Dp, H), lambda i: (0, 0)),
            pl.BlockSpec((Dp, H), lambda i: (0, 0)),
            pl.BlockSpec((H, D), lambda i: (0, 0)),
            pl.BlockSpec((1, D), lambda i: (0, 0)),
            pl.BlockSpec((1, D), lambda i: (0, 0)),
            pl.BlockSpec((1, D), lambda i: (0, 0)),
        ],
        out_specs=pl.BlockSpec((BN, D), lambda i: (i, 0)),
        out_shape=jax.ShapeDtypeStruct((N, D), jnp.float32),
    )(num2, x, xa, easum, c8.reshape(1, H), psel, padd, eexp,
      bias.reshape(1, D), ln_g.reshape(1, D), ln_b.reshape(1, D))
    return out
